# Initial kernel scaffold; baseline (speedup 1.0000x reference)
import functools

import jax
import jax.numpy as jnp
from jax import lax
from jax.experimental import pallas as pl
from jax.experimental.pallas import tpu as pltpu
from jax.experimental.pallas import tpu_sc as plsc

HEADS = 4
RR = 8
TOP_K_FRAC = 0.25


def _qkv_pool_body(x_ref, wq_ref, wk_ref, wv_ref, wck_ref, wcv_ref,
                   q_ref, ks_ref, vs_ref):
    i = pl.program_id(0)
    xb = x_ref[0]
    q_ref[0] = jnp.dot(wq_ref[...], xb, preferred_element_type=jnp.float32)
    k = jnp.dot(wk_ref[...], xb, preferred_element_type=jnp.float32)
    v = jnp.dot(wv_ref[...], xb, preferred_element_type=jnp.float32)
    kc = wck_ref[0] * k
    vc = wcv_ref[0] * v

    @pl.when(i == 0)
    def _():
        ks_ref[...] = kc
        vs_ref[...] = vc

    @pl.when(i > 0)
    def _():
        ks_ref[...] += kc
        vs_ref[...] += vc


def _score_body(q_ref, ks_ref, ts_ref, *, heads, dim, scale):
    i = pl.program_id(0)

    @pl.when(i == 0)
    def _():
        ts_ref[...] = jnp.zeros_like(ts_ref)

    qb = q_ref[0]
    for h in range(heads):
        qh = qb[h * dim:(h + 1) * dim, :]
        kh = ks_ref[h * dim:(h + 1) * dim, :]
        s = lax.dot_general(qh, kh, (((0,), (0,)), ((), ())),
                            preferred_element_type=jnp.float32) * scale
        m = jnp.max(s, axis=1, keepdims=True)
        p = jnp.exp(s - m)
        p = p / jnp.sum(p, axis=1, keepdims=True)
        ts_ref[h:h + 1, :] += jnp.sum(p, axis=0, keepdims=True)


def _attn2_body(q_ref, kt_ref, vt_ref, wout_ref, bout_ref, out_ref,
                *, heads, dim, scale):
    qb = q_ref[0]
    acc = None
    for h in range(heads):
        qh = qb[h * dim:(h + 1) * dim, :]
        kt = kt_ref[h]
        vt = vt_ref[h]
        s = lax.dot_general(qh, kt, (((0,), (1,)), ((), ())),
                            preferred_element_type=jnp.float32) * scale
        m = jnp.max(s, axis=1, keepdims=True)
        p = jnp.exp(s - m)
        p = p / jnp.sum(p, axis=1, keepdims=True)
        oh = lax.dot_general(p, vt, (((1,), (0,)), ((), ())),
                             preferred_element_type=jnp.float32)
        wo_h = wout_ref[:, h * dim:(h + 1) * dim]
        c = lax.dot_general(wo_h, oh, (((1,), (1,)), ((), ())),
                            preferred_element_type=jnp.float32)
        acc = c if acc is None else acc + c
    out_ref[0] = acc + bout_ref[...]


def _make_select_gather(heads, ntok, topk, dim):
    nchunk = ntok // 16
    p0 = 104
    p1 = topk - p0
    pad = 112
    mesh = plsc.VectorSubcoreMesh(core_axis_name="c", subcore_axis_name="s")

    @functools.partial(
        pl.kernel, mesh=mesh,
        out_type=[jax.ShapeDtypeStruct((heads * topk, dim), jnp.float32),
                  jax.ShapeDtypeStruct((heads * topk, dim), jnp.float32)],
        scratch_types=[pltpu.VMEM((ntok,), jnp.float32),
                       pltpu.VMEM((pad,), jnp.int32),
                       pltpu.VMEM((pad,), jnp.int32),
                       pltpu.VMEM((pad, dim), jnp.float32),
                       pltpu.VMEM((pad, dim), jnp.float32),
                       pltpu.SemaphoreType.DMA],
    )
    def sel_kernel(ts_hbm, k_hbm, v_hbm, kt_hbm, vt_hbm,
                   sc_v, idx0_v, idx1_v, rows0_v, rows1_v, sem):
        wid = lax.axis_index("s") * 2 + lax.axis_index("c")

        @pl.when(wid < heads)
        def _():
            head = wid
            pltpu.sync_copy(ts_hbm.at[head], sc_v)

            zi = jnp.zeros((16,), jnp.int32)

            def cnt_ge(mid):
                def body(j, acc):
                    sv = plsc.bitcast(sc_v[pl.ds(j * 16, 16)], jnp.int32)
                    return acc + plsc.all_reduce_population_count(sv >= mid)
                return lax.fori_loop(0, nchunk, body, zi)

            def sbody(_, carry):
                lo, hi = carry
                mid = lo + ((hi - lo) >> 1)
                ge = cnt_ge(mid) >= topk
                return jnp.where(ge, mid, lo), jnp.where(ge, hi, mid)

            lo0 = jnp.zeros((16,), jnp.int32)
            hi0 = jnp.full((16,), 0x7F800000, jnp.int32)
            tau, _ = lax.fori_loop(0, 31, sbody, (lo0, hi0))
            need_eq = jnp.full((16,), topk, jnp.int32) - cnt_ge(tau + 1)

            for t in range(pad // 16):
                idx0_v[pl.ds(t * 16, 16)] = zi
                idx1_v[pl.ds(t * 16, 16)] = zi

            def ibody(j, carry):
                sel_cnt, eq_cnt = carry
                base = j * 16
                sv = plsc.bitcast(sc_v[pl.ds(base, 16)], jnp.int32)
                idxs = lax.iota(jnp.int32, 16) + base + head * ntok
                gt = sv > tau
                eq = sv == tau
                eq_i = jnp.where(eq, 1, 0)
                eq_pref = plsc.cumsum(eq_i) - eq_i + eq_cnt
                sel = gt | (eq & (eq_pref < need_eq))
                sel_i = jnp.where(sel, 1, 0)
                pos = plsc.cumsum(sel_i) - sel_i + sel_cnt
                in0 = sel & (pos < p0)
                in1 = sel & (pos >= p0)
                plsc.store_scatter(idx0_v, [jnp.where(in0, pos, 0)], idxs, in0)
                plsc.store_scatter(idx1_v, [jnp.where(in1, pos - p0, 0)],
                                   idxs, in1)
                sel_cnt = sel_cnt + plsc.all_reduce_population_count(sel)
                eq_cnt = eq_cnt + plsc.all_reduce_population_count(eq)
                return sel_cnt, eq_cnt

            lax.fori_loop(0, nchunk, ibody, (zi, zi))

            pltpu.async_copy(k_hbm.at[idx0_v], rows0_v, sem).wait()
            pltpu.sync_copy(rows0_v.at[pl.ds(0, p0)],
                            kt_hbm.at[pl.ds(head * topk, p0)])
            pltpu.async_copy(k_hbm.at[idx1_v], rows0_v, sem).wait()
            pltpu.sync_copy(rows0_v.at[pl.ds(0, p1)],
                            kt_hbm.at[pl.ds(head * topk + p0, p1)])
            pltpu.async_copy(v_hbm.at[idx0_v], rows1_v, sem).wait()
            pltpu.sync_copy(rows1_v.at[pl.ds(0, p0)],
                            vt_hbm.at[pl.ds(head * topk, p0)])
            pltpu.async_copy(v_hbm.at[idx1_v], rows1_v, sem).wait()
            pltpu.sync_copy(rows1_v.at[pl.ds(0, p1)],
                            vt_hbm.at[pl.ds(head * topk + p0, p1)])

    return sel_kernel


def kernel(x, Wq, Wk, Wv, Wck, Wcv, Wout, b_out):
    B, C, H, W = x.shape
    heads = HEADS
    dim = C // heads
    scale = dim ** (-0.5)
    r = RR
    Hs, Ws = H // r, W // r
    n = Hs * Ws
    rr = r * r
    topk = max(1, int(n * TOP_K_FRAC))

    f32 = jnp.float32
    x = x.astype(f32)

    xr = x.reshape(C, Hs, r, Ws, r).transpose(2, 4, 0, 1, 3).reshape(rr, C, n)
    wck_r = Wck.reshape(C, rr).T.reshape(rr, C, 1)
    wcv_r = Wcv.reshape(C, rr).T.reshape(rr, C, 1)

    q, ks, vs = pl.pallas_call(
        _qkv_pool_body,
        grid=(rr,),
        in_specs=[
            pl.BlockSpec((1, C, n), lambda i: (i, 0, 0)),
            pl.BlockSpec((C, C), lambda i: (0, 0)),
            pl.BlockSpec((C, C), lambda i: (0, 0)),
            pl.BlockSpec((C, C), lambda i: (0, 0)),
            pl.BlockSpec((1, C, 1), lambda i: (i, 0, 0)),
            pl.BlockSpec((1, C, 1), lambda i: (i, 0, 0)),
        ],
        out_specs=[
            pl.BlockSpec((1, C, n), lambda i: (i, 0, 0)),
            pl.BlockSpec((C, n), lambda i: (0, 0)),
            pl.BlockSpec((C, n), lambda i: (0, 0)),
        ],
        out_shape=[
            jax.ShapeDtypeStruct((rr, C, n), f32),
            jax.ShapeDtypeStruct((C, n), f32),
            jax.ShapeDtypeStruct((C, n), f32),
        ],
        compiler_params=pltpu.CompilerParams(
            dimension_semantics=("arbitrary",)),
    )(xr, Wq, Wk, Wv, wck_r, wcv_r)

    ts = pl.pallas_call(
        functools.partial(_score_body, heads=heads, dim=dim, scale=scale),
        grid=(rr,),
        in_specs=[
            pl.BlockSpec((1, C, n), lambda i: (i, 0, 0)),
            pl.BlockSpec((C, n), lambda i: (0, 0)),
        ],
        out_specs=pl.BlockSpec((heads, n), lambda i: (0, 0)),
        out_shape=jax.ShapeDtypeStruct((heads, n), f32),
        compiler_params=pltpu.CompilerParams(
            dimension_semantics=("arbitrary",)),
    )(q, ks)

    k2t = ks.reshape(heads, dim, n).transpose(0, 2, 1).reshape(heads * n, dim)
    v2t = vs.reshape(heads, dim, n).transpose(0, 2, 1).reshape(heads * n, dim)

    sel = _make_select_gather(heads, n, topk, dim)
    kt_flat, vt_flat = sel(ts, k2t, v2t)
    kt = kt_flat.reshape(heads, topk, dim)
    vt = vt_flat.reshape(heads, topk, dim)

    outr = pl.pallas_call(
        functools.partial(_attn2_body, heads=heads, dim=dim, scale=scale),
        grid=(rr,),
        in_specs=[
            pl.BlockSpec((1, C, n), lambda i: (i, 0, 0)),
            pl.BlockSpec((heads, topk, dim), lambda i: (0, 0, 0)),
            pl.BlockSpec((heads, topk, dim), lambda i: (0, 0, 0)),
            pl.BlockSpec((C, C), lambda i: (0, 0)),
            pl.BlockSpec((C, 1), lambda i: (0, 0)),
        ],
        out_specs=pl.BlockSpec((1, C, n), lambda i: (i, 0, 0)),
        out_shape=jax.ShapeDtypeStruct((rr, C, n), f32),
        compiler_params=pltpu.CompilerParams(
            dimension_semantics=("arbitrary",)),
    )(q, kt, vt, Wout, b_out.reshape(C, 1))

    out = outr.reshape(r, r, C, Hs, Ws).transpose(2, 3, 0, 4, 1)
    return out.reshape(B, C, H, W)


# speedup vs baseline: 1.0079x; 1.0079x over previous
import functools

import jax
import jax.numpy as jnp
from jax import lax
from jax.experimental import pallas as pl
from jax.experimental.pallas import tpu as pltpu
from jax.experimental.pallas import tpu_sc as plsc

HEADS = 4
RR = 8
TOP_K_FRAC = 0.25


def _bdot(a, b, dims):
    return lax.dot_general(a.astype(jnp.bfloat16), b.astype(jnp.bfloat16),
                           dims, preferred_element_type=jnp.float32)


def _qkv_pool_body(x_ref, wq_ref, wk_ref, wv_ref, wck_ref, wcv_ref,
                   q_ref, ks_ref, vs_ref):
    i = pl.program_id(0)
    xb = x_ref[0]
    q_ref[0] = _bdot(wq_ref[...], xb, (((1,), (0,)), ((), ())))
    k = _bdot(wk_ref[...], xb, (((1,), (0,)), ((), ())))
    v = _bdot(wv_ref[...], xb, (((1,), (0,)), ((), ())))
    kc = wck_ref[0] * k
    vc = wcv_ref[0] * v

    @pl.when(i == 0)
    def _():
        ks_ref[...] = kc
        vs_ref[...] = vc

    @pl.when(i > 0)
    def _():
        ks_ref[...] += kc
        vs_ref[...] += vc


def _score_body(q_ref, ks_ref, ts_ref, acc_ref, *, heads, dim, scale):
    i = pl.program_id(0)
    nq = q_ref.shape[1]
    ngrp = nq // 128

    @pl.when(i == 0)
    def _():
        acc_ref[...] = jnp.zeros_like(acc_ref)

    qb = q_ref[...].astype(jnp.bfloat16)
    for h in range(heads):
        qh = qb[h * dim:(h + 1) * dim, :]
        kh = ks_ref[h * dim:(h + 1) * dim, :]
        sT = lax.dot_general(kh, qh, (((0,), (0,)), ((), ())),
                             preferred_element_type=jnp.float32) * scale
        m = jnp.max(sT, axis=0, keepdims=True)
        e = jnp.exp(sT - m)
        den = jnp.sum(e, axis=0, keepdims=True)
        p = e / den
        for g in range(ngrp):
            acc_ref[h] += p[:, g * 128:(g + 1) * 128]

    @pl.when(i == pl.num_programs(0) - 1)
    def _():
        for h in range(heads):
            ts_ref[h] = jnp.sum(acc_ref[h], axis=1, keepdims=True)


def _attn2_body(q_ref, kt_ref, vt_ref, wout_ref, bout_ref, out_ref,
                *, heads, dim, scale):
    qb = q_ref[...]
    acc = None
    for h in range(heads):
        qh = qb[h * dim:(h + 1) * dim, :]
        kt = kt_ref[h]
        vt = vt_ref[h]
        s = _bdot(qh, kt, (((0,), (1,)), ((), ()))) * scale
        m = jnp.max(s, axis=1, keepdims=True)
        p = jnp.exp(s - m)
        p = p / jnp.sum(p, axis=1, keepdims=True)
        oh = _bdot(p, vt, (((1,), (0,)), ((), ())))
        wo_h = wout_ref[:, h * dim:(h + 1) * dim]
        c = _bdot(wo_h, oh, (((1,), (1,)), ((), ())))
        acc = c if acc is None else acc + c
    out_ref[...] = acc + bout_ref[...]


def _make_select_gather(heads, ntok, topk, gdim):
    nchunk = ntok // 16
    p0 = 104
    p1 = topk - p0
    pad = 112
    mesh = plsc.VectorSubcoreMesh(core_axis_name="c", subcore_axis_name="s")

    @functools.partial(
        pl.kernel, mesh=mesh,
        compiler_params=pltpu.CompilerParams(needs_layout_passes=False),
        out_type=[jax.ShapeDtypeStruct((heads, topk, gdim), jnp.float32),
                  jax.ShapeDtypeStruct((heads, topk, gdim), jnp.float32)],
        scratch_types=[pltpu.VMEM((ntok,), jnp.int32),
                       pltpu.VMEM((pad,), jnp.int32),
                       pltpu.VMEM((pad,), jnp.int32),
                       pltpu.VMEM((pad, gdim), jnp.float32),
                       pltpu.VMEM((pad, gdim), jnp.float32),
                       pltpu.SemaphoreType.DMA],
    )
    def sel_kernel(ts_hbm, k_hbm, v_hbm, kt_hbm, vt_hbm,
                   sc_v, idx0_v, idx1_v, rows0_v, rows1_v, sem):
        wid = lax.axis_index("s") * 2 + lax.axis_index("c")

        @pl.when(wid < heads)
        def _():
            head = wid
            pltpu.sync_copy(ts_hbm.at[head], sc_v)

            zi = jnp.zeros((16,), jnp.int32)

            def cnt_ge(mid):
                def body(j, acc):
                    sv = sc_v[pl.ds(j * 16, 16)]
                    return acc + plsc.all_reduce_population_count(sv >= mid)
                return lax.fori_loop(0, nchunk, body, zi)

            def sbody(_, carry):
                lo, hi = carry
                mid = lo + ((hi - lo) >> 1)
                ge = cnt_ge(mid) >= topk
                return jnp.where(ge, mid, lo), jnp.where(ge, hi, mid)

            lo0 = jnp.zeros((16,), jnp.int32)
            hi0 = jnp.full((16,), 0x7F800000, jnp.int32)
            tau, _ = lax.fori_loop(0, 31, sbody, (lo0, hi0))
            need_eq = jnp.full((16,), topk, jnp.int32) - cnt_ge(tau + 1)

            for t in range(pad // 16):
                idx0_v[pl.ds(t * 16, 16)] = zi
                idx1_v[pl.ds(t * 16, 16)] = zi

            def ibody(j, carry):
                sel_cnt, eq_cnt = carry
                base = j * 16
                sv = sc_v[pl.ds(base, 16)]
                idxs = lax.iota(jnp.int32, 16) + base + head * ntok
                gt = sv > tau
                eq = sv == tau
                eq_i = jnp.where(eq, 1, 0)
                eq_pref = plsc.cumsum(eq_i) - eq_i + eq_cnt
                sel = gt | (eq & (eq_pref < need_eq))
                sel_i = jnp.where(sel, 1, 0)
                pos = plsc.cumsum(sel_i) - sel_i + sel_cnt
                in0 = sel & (pos < p0)
                in1 = sel & (pos >= p0)
                plsc.store_scatter(idx0_v, [jnp.where(in0, pos, 0)], idxs,
                                   mask=in0)
                plsc.store_scatter(idx1_v, [jnp.where(in1, pos - p0, 0)],
                                   idxs, mask=in1)
                sel_cnt = sel_cnt + plsc.all_reduce_population_count(sel)
                eq_cnt = eq_cnt + plsc.all_reduce_population_count(eq)
                return sel_cnt, eq_cnt

            lax.fori_loop(0, nchunk, ibody, (zi, zi))

            pltpu.async_copy(k_hbm.at[idx0_v], rows0_v, sem).wait()
            pltpu.sync_copy(rows0_v.at[pl.ds(0, p0)],
                            kt_hbm.at[head].at[pl.ds(0, p0)])
            pltpu.async_copy(k_hbm.at[idx1_v], rows0_v, sem).wait()
            pltpu.sync_copy(rows0_v.at[pl.ds(0, p1)],
                            kt_hbm.at[head].at[pl.ds(p0, p1)])
            pltpu.async_copy(v_hbm.at[idx0_v], rows1_v, sem).wait()
            pltpu.sync_copy(rows1_v.at[pl.ds(0, p0)],
                            vt_hbm.at[head].at[pl.ds(0, p0)])
            pltpu.async_copy(v_hbm.at[idx1_v], rows1_v, sem).wait()
            pltpu.sync_copy(rows1_v.at[pl.ds(0, p1)],
                            vt_hbm.at[head].at[pl.ds(p0, p1)])

    return sel_kernel


def kernel(x, Wq, Wk, Wv, Wck, Wcv, Wout, b_out):
    B, C, H, W = x.shape
    heads = HEADS
    dim = C // heads
    scale = dim ** (-0.5)
    r = RR
    Hs, Ws = H // r, W // r
    n = Hs * Ws
    rr = r * r
    N = H * W
    topk = max(1, int(n * TOP_K_FRAC))

    f32 = jnp.float32
    x = x.astype(f32)

    xr = x.reshape(C, Hs, r, Ws, r).transpose(2, 4, 0, 1, 3).reshape(rr, C, n)
    wck_r = Wck.reshape(C, rr).T.reshape(rr, C, 1)
    wcv_r = Wcv.reshape(C, rr).T.reshape(rr, C, 1)

    q, ks, vs = pl.pallas_call(
        _qkv_pool_body,
        grid=(rr,),
        in_specs=[
            pl.BlockSpec((1, C, n), lambda i: (i, 0, 0)),
            pl.BlockSpec((C, C), lambda i: (0, 0)),
            pl.BlockSpec((C, C), lambda i: (0, 0)),
            pl.BlockSpec((C, C), lambda i: (0, 0)),
            pl.BlockSpec((1, C, 1), lambda i: (i, 0, 0)),
            pl.BlockSpec((1, C, 1), lambda i: (i, 0, 0)),
        ],
        out_specs=[
            pl.BlockSpec((1, C, n), lambda i: (i, 0, 0)),
            pl.BlockSpec((C, n), lambda i: (0, 0)),
            pl.BlockSpec((C, n), lambda i: (0, 0)),
        ],
        out_shape=[
            jax.ShapeDtypeStruct((rr, C, n), f32),
            jax.ShapeDtypeStruct((C, n), f32),
            jax.ShapeDtypeStruct((C, n), f32),
        ],
        compiler_params=pltpu.CompilerParams(
            dimension_semantics=("arbitrary",)),
    )(xr, Wq, Wk, Wv, wck_r, wcv_r)

    qn = q.reshape(r, r, C, Hs, Ws).transpose(2, 3, 0, 4, 1).reshape(C, N)
    ksb = ks.astype(jnp.bfloat16)
    vsb = vs.astype(jnp.bfloat16)

    nq = 1792
    nblk = N // nq
    ts3 = pl.pallas_call(
        functools.partial(_score_body, heads=heads, dim=dim, scale=scale),
        grid=(nblk,),
        in_specs=[
            pl.BlockSpec((C, nq), lambda i: (0, i)),
            pl.BlockSpec((C, n), lambda i: (0, 0)),
        ],
        out_specs=pl.BlockSpec((heads, n, 1), lambda i: (0, 0, 0)),
        out_shape=jax.ShapeDtypeStruct((heads, n, 1), f32),
        scratch_shapes=[pltpu.VMEM((heads, n, 128), f32)],
        compiler_params=pltpu.CompilerParams(
            dimension_semantics=("arbitrary",)),
    )(qn, ksb)
    ts = ts3[:, :, 0]

    gdim = 128
    k2t = (ksb.astype(f32).reshape(heads, dim, n)
           .transpose(0, 2, 1).reshape(heads * n, dim))
    v2t = (vsb.astype(f32).reshape(heads, dim, n)
           .transpose(0, 2, 1).reshape(heads * n, dim))
    k2t = jnp.pad(k2t, ((0, 0), (0, gdim - dim)))
    v2t = jnp.pad(v2t, ((0, 0), (0, gdim - dim)))

    ts_bits = lax.bitcast_convert_type(ts, jnp.int32)
    sel = _make_select_gather(heads, n, topk, gdim)
    kt, vt = sel(ts_bits, k2t, v2t)
    kt = kt[:, :, :dim]
    vt = vt[:, :, :dim]

    out = pl.pallas_call(
        functools.partial(_attn2_body, heads=heads, dim=dim, scale=scale),
        grid=(nblk,),
        in_specs=[
            pl.BlockSpec((C, nq), lambda i: (0, i)),
            pl.BlockSpec((heads, topk, dim), lambda i: (0, 0, 0)),
            pl.BlockSpec((heads, topk, dim), lambda i: (0, 0, 0)),
            pl.BlockSpec((C, C), lambda i: (0, 0)),
            pl.BlockSpec((C, 1), lambda i: (0, 0)),
        ],
        out_specs=pl.BlockSpec((C, nq), lambda i: (0, i)),
        out_shape=jax.ShapeDtypeStruct((C, N), f32),
        compiler_params=pltpu.CompilerParams(
            dimension_semantics=("arbitrary",)),
    )(qn, kt, vt, Wout, b_out.reshape(C, 1))

    return out.reshape(B, C, H, W)
